# baseline (speedup 1.0000x reference)
import jax
import jax.numpy as jnp
from jax import lax
from jax.experimental import pallas as pl
from jax.experimental.pallas import tpu as pltpu


_ROWS = 4096
_COLS = 512
_BLK = 1024
_GRID = _ROWS // _BLK
_GROUPS = _ROWS // 256
_SLABS = _BLK // 256
_N = float(_ROWS * _COLS)


def _body(x_ref, t_ref, w_ref, o_ref, acc_ref):
    i = pl.program_id(0)

    @pl.when(i == 0)
    def _init():
        for k in range(5):
            acc_ref[k] = 0.0

    w = w_ref[...]
    s = [jnp.zeros((256, _COLS), jnp.float32) for _ in range(5)]
    one = jnp.ones((256, _COLS), jnp.float32)
    for half in range(_SLABS):
        r0 = 256 * half
        d = jnp.abs(x_ref[r0:r0 + 256, :] - t_ref[r0:r0 + 256, :])
        g = _SLABS * i + half
        pm = lax.shift_left(w, 31 - 2 * g) < 0
        ph = lax.shift_left(w, 30 - 2 * g) < 0
        zero = jnp.zeros_like(d)
        s[0] = s[0] + jnp.where(pm, d, zero)
        s[1] = s[1] + jnp.where(pm, one, zero)
        s[2] = s[2] + jnp.where(ph, d, zero)
        s[3] = s[3] + jnp.where(ph, one, zero)
        s[4] = s[4] + d
    for k in range(5):
        acc_ref[k] += jnp.sum(s[k])

    @pl.when(i == pl.num_programs(0) - 1)
    def _fin():
        s_m, c_m, s_h, c_h, s_t = (acc_ref[0], acc_ref[1], acc_ref[2],
                                   acc_ref[3], acc_ref[4])
        o_ref[0] = (s_m / c_m + s_h / c_h + (s_t - s_h) / (_N - c_h)) / 3.0


def _pack(masks, hull):
    m3 = masks.reshape(_GROUPS, 256, _COLS)
    h3 = hull.reshape(_GROUPS, 256, _COLS)
    planes = []
    for j in range(4):
        terms = []
        for q in range(4):
            k = 4 * j + q
            terms.append(m3[k].astype(jnp.uint8) << (2 * q))
            terms.append(h3[k].astype(jnp.uint8) << (2 * q + 1))
        while len(terms) > 1:
            terms = [a | b for a, b in zip(terms[::2], terms[1::2])]
        planes.append(terms[0].astype(jnp.uint32) << (8 * j))
    w = (planes[0] | planes[1]) | (planes[2] | planes[3])
    return lax.bitcast_convert_type(w, jnp.int32)


def kernel(input, target, masks, hull):
    x = input.reshape(_ROWS, _COLS)
    t = target.reshape(_ROWS, _COLS)
    w = _pack(masks, hull)

    spec = pl.BlockSpec((_BLK, _COLS), lambda i: (i, 0))
    wspec = pl.BlockSpec((256, _COLS), lambda i: (0, 0))
    out = pl.pallas_call(
        _body,
        grid=(_GRID,),
        in_specs=[spec, spec, wspec],
        out_specs=pl.BlockSpec(memory_space=pltpu.SMEM),
        out_shape=jax.ShapeDtypeStruct((1,), jnp.float32),
        scratch_shapes=[pltpu.SMEM((5,), jnp.float32)],
    )(x, t, w)
    return out[0]


# speedup vs baseline: 1.0232x; 1.0232x over previous
import jax
import jax.numpy as jnp
from jax import lax
from jax.experimental import pallas as pl
from jax.experimental.pallas import tpu as pltpu


_ROWS = 4096
_COLS = 512
_BLK = 1024
_GRID = _ROWS // _BLK
_GROUPS = _ROWS // 256
_SLABS = _BLK // 256
_N = float(_ROWS * _COLS)


def _body(x_ref, t_ref, w_ref, o_ref, acc_ref):
    i = pl.program_id(0)

    @pl.when(i == 0)
    def _init():
        for k in range(5):
            acc_ref[k] = 0.0

    w = w_ref[...]
    s = [jnp.zeros((256, _COLS), jnp.float32) for _ in range(5)]
    one = jnp.ones((256, _COLS), jnp.float32)
    for half in range(_SLABS):
        r0 = 256 * half
        d = jnp.abs(x_ref[r0:r0 + 256, :] - t_ref[r0:r0 + 256, :])
        g = _SLABS * i + half
        pm = lax.shift_left(w, 31 - 2 * g) < 0
        ph = lax.shift_left(w, 30 - 2 * g) < 0
        zero = jnp.zeros_like(d)
        s[0] = s[0] + jnp.where(pm, d, zero)
        s[1] = s[1] + jnp.where(pm, one, zero)
        s[2] = s[2] + jnp.where(ph, d, zero)
        s[3] = s[3] + jnp.where(ph, one, zero)
        s[4] = s[4] + d
    for k in range(5):
        acc_ref[k] += jnp.sum(s[k])

    @pl.when(i == pl.num_programs(0) - 1)
    def _fin():
        s_m, c_m, s_h, c_h, s_t = (acc_ref[0], acc_ref[1], acc_ref[2],
                                   acc_ref[3], acc_ref[4])
        o_ref[0] = (s_m / c_m + s_h / c_h + (s_t - s_h) / (_N - c_h)) / 3.0


def _pack(masks, hull):
    m3 = masks.reshape(_GROUPS, 256, _COLS)
    h3 = hull.reshape(_GROUPS, 256, _COLS)
    terms = []
    for k in range(_GROUPS):
        terms.append(m3[k].astype(jnp.uint32) << (2 * k))
        terms.append(h3[k].astype(jnp.uint32) << (2 * k + 1))
    while len(terms) > 1:
        terms = [a | b for a, b in zip(terms[::2], terms[1::2])]
    return lax.bitcast_convert_type(terms[0], jnp.int32)


def kernel(input, target, masks, hull):
    x = input.reshape(_ROWS, _COLS)
    t = target.reshape(_ROWS, _COLS)
    w = _pack(masks, hull)

    spec = pl.BlockSpec((_BLK, _COLS), lambda i: (i, 0))
    wspec = pl.BlockSpec((256, _COLS), lambda i: (0, 0))
    out = pl.pallas_call(
        _body,
        grid=(_GRID,),
        in_specs=[spec, spec, wspec],
        out_specs=pl.BlockSpec(memory_space=pltpu.SMEM),
        out_shape=jax.ShapeDtypeStruct((1,), jnp.float32),
        scratch_shapes=[pltpu.SMEM((5,), jnp.float32)],
    )(x, t, w)
    return out[0]
